# Initial kernel scaffold; baseline (speedup 1.0000x reference)
import jax
import jax.numpy as jnp
from jax.experimental import pallas as pl


def kernel(x, sql, sql_table, input_table, gw1, gb1, gw2, gb2, ew1, eb1, ew2, eb2):
    raise NotImplementedError("write your pallas kernel here")


# speedup vs baseline: 3.7583x; 3.7583x over previous
import functools

import jax
import jax.numpy as jnp
from jax import lax
from jax.experimental import pallas as pl
from jax.experimental.pallas import tpu as pltpu
from jax.experimental.pallas import tpu_sc as plsc

NFIELD = 26
SQL_NEMB = 16
DATA_NEMB = 16
K = 16
HID = 64
B = 4096

NC, NS = 2, 16
NW = NC * NS
ROWS = B * NFIELD
CH = 128
NCHUNK = ROWS // (NW * CH)


def _sc_gather(xi, si, xtab, stab):
    mesh = plsc.VectorSubcoreMesh(
        core_axis_name="c", subcore_axis_name="s",
        num_cores=NC, num_subcores=NS)

    @functools.partial(
        pl.kernel,
        out_type=(
            jax.ShapeDtypeStruct((NW, NCHUNK, CH, DATA_NEMB), jnp.float32),
            jax.ShapeDtypeStruct((NW, NCHUNK, CH, SQL_NEMB), jnp.float32),
        ),
        mesh=mesh,
        scratch_types=[
            pltpu.VMEM((NCHUNK, CH), jnp.int32),
            pltpu.VMEM((NCHUNK, CH), jnp.int32),
            pltpu.VMEM((NCHUNK, CH, DATA_NEMB), jnp.float32),
            pltpu.VMEM((NCHUNK, CH, SQL_NEMB), jnp.float32),
            pltpu.SemaphoreType.DMA,
            pltpu.SemaphoreType.DMA,
        ],
        compiler_params=pltpu.CompilerParams(use_tc_tiling_on_sc=False),
    )
    def gather_kernel(xi_hbm, si_hbm, xtab_hbm, stab_hbm, xout_hbm, sout_hbm,
                      xi_v, si_v, xr_v, sr_v, sem_x, sem_s):
        wid = lax.axis_index("s") * NC + lax.axis_index("c")
        pltpu.sync_copy(xi_hbm.at[wid], xi_v)
        pltpu.sync_copy(si_hbm.at[wid], si_v)

        @pl.loop(0, NCHUNK)
        def _fire(j):
            pltpu.async_copy(xtab_hbm.at[xi_v.at[j]], xr_v.at[j], sem_x)
            pltpu.async_copy(stab_hbm.at[si_v.at[j]], sr_v.at[j], sem_s)

        pltpu.make_async_copy(xout_hbm.at[wid], xr_v, sem_x).wait()
        pltpu.make_async_copy(sout_hbm.at[wid], sr_v, sem_s).wait()
        pltpu.sync_copy(xr_v, xout_hbm.at[wid])
        pltpu.sync_copy(sr_v, sout_hbm.at[wid])

    return gather_kernel(xi, si, xtab, stab)


def _tc_fused(x_emb, sql_emb, gw1, gb1, gw2, gb2, w1c, b1c, w2blk, eb2r):
    T = 8
    BT = B // T
    EXP_IN = NFIELD * DATA_NEMB
    GATE_IN = NFIELD * SQL_NEMB
    KH = K * HID

    def body(xe_ref, se_ref, gw1_ref, gb1_ref, gw2_ref, gb2_ref,
             w1c_ref, b1c_ref, w2_ref, eb2_ref, y_ref, loss_ref, acc_ref):
        t = pl.program_id(0)

        se = se_ref[...]
        gh = jnp.maximum(
            jnp.dot(se, gw1_ref[...], preferred_element_type=jnp.float32)
            + gb1_ref[...], 0.0)
        gl = (jnp.dot(gh, gw2_ref[...], preferred_element_type=jnp.float32)
              + gb2_ref[...])
        gm = jnp.max(gl, axis=-1, keepdims=True)
        ge = jnp.exp(gl - gm)
        p = ge / jnp.sum(ge, axis=-1, keepdims=True)

        col = lax.broadcasted_iota(jnp.int32, p.shape, 1)
        v1 = jnp.max(p, axis=-1, keepdims=True)
        i1 = jnp.min(jnp.where(p == v1, col, K), axis=-1, keepdims=True)
        mask1 = col == i1
        pm = jnp.where(mask1, -1.0, p)
        v2 = jnp.max(pm, axis=-1, keepdims=True)
        i2 = jnp.min(jnp.where(pm == v2, col, K), axis=-1, keepdims=True)
        mask2 = col == i2
        gates = (jnp.where(mask1, v1, 0.0)
                 + jnp.where(mask2, v2, 0.0))

        xe = xe_ref[...]
        h = jnp.maximum(
            jnp.dot(xe, w1c_ref[...], preferred_element_type=jnp.float32)
            + b1c_ref[...], 0.0)
        eo = (jnp.dot(h, w2_ref[...], preferred_element_type=jnp.float32)
              + eb2_ref[...])
        y_ref[...] = jnp.sum(gates * eo, axis=-1, keepdims=True)

        @pl.when(t == 0)
        def _init():
            acc_ref[...] = jnp.zeros_like(acc_ref)

        acc_ref[...] += jnp.sum(gates, axis=0, keepdims=True)

        @pl.when(t == T - 1)
        def _fin():
            imp = acc_ref[...]
            mean = jnp.sum(imp, axis=-1, keepdims=True) / K
            var = jnp.sum((imp - mean) ** 2, axis=-1, keepdims=True) / K
            loss_ref[...] = var / (mean * mean + 1e-10)

    y, loss = pl.pallas_call(
        body,
        grid=(T,),
        in_specs=[
            pl.BlockSpec((BT, EXP_IN), lambda t: (t, 0)),
            pl.BlockSpec((BT, GATE_IN), lambda t: (t, 0)),
            pl.BlockSpec((GATE_IN, HID), lambda t: (0, 0)),
            pl.BlockSpec((1, HID), lambda t: (0, 0)),
            pl.BlockSpec((HID, K), lambda t: (0, 0)),
            pl.BlockSpec((1, K), lambda t: (0, 0)),
            pl.BlockSpec((EXP_IN, KH), lambda t: (0, 0)),
            pl.BlockSpec((1, KH), lambda t: (0, 0)),
            pl.BlockSpec((KH, K), lambda t: (0, 0)),
            pl.BlockSpec((1, K), lambda t: (0, 0)),
        ],
        out_specs=[
            pl.BlockSpec((BT, 1), lambda t: (t, 0)),
            pl.BlockSpec((1, 1), lambda t: (0, 0)),
        ],
        out_shape=[
            jax.ShapeDtypeStruct((B, 1), jnp.float32),
            jax.ShapeDtypeStruct((1, 1), jnp.float32),
        ],
        scratch_shapes=[pltpu.VMEM((1, K), jnp.float32)],
    )(x_emb, sql_emb, gw1, gb1, gw2, gb2, w1c, b1c, w2blk, eb2r)
    return y, loss


def kernel(x, sql, sql_table, input_table, gw1, gb1, gw2, gb2, ew1, eb1, ew2, eb2):
    xi = x.reshape(NW, NCHUNK, CH)
    si = sql.reshape(NW, NCHUNK, CH)
    xr, sr = _sc_gather(xi, si, input_table, sql_table)
    x_emb = xr.reshape(B, NFIELD * DATA_NEMB)
    sql_emb = sr.reshape(B, NFIELD * SQL_NEMB)

    w1c = ew1.transpose(1, 0, 2).reshape(NFIELD * DATA_NEMB, K * HID)
    b1c = eb1.reshape(1, K * HID)
    eyek = jnp.eye(K, dtype=jnp.float32)
    w2blk = (ew2[:, :, 0][:, :, None] * eyek[:, None, :]).reshape(K * HID, K)
    eb2r = eb2.reshape(1, K)

    y, loss = _tc_fused(x_emb, sql_emb, gw1, gb1.reshape(1, HID),
                        gw2, gb2.reshape(1, K), w1c, b1c, w2blk, eb2r)
    return (y.reshape(B), loss.reshape(()))
